# baseline (speedup 1.0000x reference)
import math

import jax
import numpy as np
import jax.numpy as jnp
from jax.experimental import pallas as pl
from jax.experimental.pallas import tpu as pltpu

B, C, D, H, L = 2, 2048, 768, 12, 2
E, TOPK, FF, V = 8, 2, 3072, 32000
DH = D // H
T = B * C
CAP = (TOPK * T) // E
EPS = 1e-5
SCALE = float(np.sqrt(np.float32(D)))
GR = 8
TB = 256
CQ = 512
CA = 2048
CB = 256
KC = 256
INV_PAD = E * CAP + 8


def _gather_krn(idx_ref, *refs):
    del idx_ref
    ins, out_ref = refs[:GR], refs[GR]
    out_ref[...] = jnp.concatenate([r[0] for r in ins], axis=0)


def _row_gather(table, idx, n_rows, clip_hi):
    def imap(j):
        def f(i, idx_ref):
            v = idx_ref[i * GR + j]
            return (jnp.minimum(jnp.maximum(v, 0), clip_hi), 0, 0)
        return f

    grid_spec = pltpu.PrefetchScalarGridSpec(
        num_scalar_prefetch=1,
        grid=(n_rows // GR,),
        in_specs=[pl.BlockSpec((1, 1, D), imap(j)) for j in range(GR)],
        out_specs=pl.BlockSpec((GR, D), lambda i, idx_ref: (i, 0)),
    )
    return pl.pallas_call(
        _gather_krn,
        grid_spec=grid_spec,
        out_shape=jax.ShapeDtypeStruct((n_rows, D), jnp.float32),
        compiler_params=pltpu.CompilerParams(
            dimension_semantics=("arbitrary",)),
    )(idx, *([table.reshape(-1, 1, D)] * GR))


def _embed_krn(idx_ref, *refs):
    del idx_ref
    toks, pos_ref, step_ref, out_ref = refs[:GR], refs[GR], refs[GR + 1], refs[GR + 2]
    rows = jnp.concatenate([r[0] for r in toks], axis=0)
    out_ref[...] = rows * SCALE + pos_ref[...] + step_ref[0]


def _embed(src_flat, tok_emb, pos_c, step_add):
    def imap(j):
        def f(i, idx_ref):
            return (idx_ref[i * GR + j], 0, 0)
        return f

    cpg = C // GR
    grid_spec = pltpu.PrefetchScalarGridSpec(
        num_scalar_prefetch=1,
        grid=(T // GR,),
        in_specs=[pl.BlockSpec((1, 1, D), imap(j)) for j in range(GR)]
        + [pl.BlockSpec((GR, D), lambda i, idx_ref: (i % cpg, 0)),
           pl.BlockSpec((1, 1, D), lambda i, idx_ref: (i // cpg, 0, 0))],
        out_specs=pl.BlockSpec((GR, D), lambda i, idx_ref: (i, 0)),
    )
    return pl.pallas_call(
        _embed_krn,
        grid_spec=grid_spec,
        out_shape=jax.ShapeDtypeStruct((T, D), jnp.float32),
        compiler_params=pltpu.CompilerParams(
            dimension_semantics=("arbitrary",)),
    )(src_flat, *([tok_emb.reshape(V, 1, D)] * GR), pos_c,
      step_add.reshape(B, 1, D))


def _qkv_krn(x_ref, wq_ref, wk_ref, wv_ref, bq_ref, bk_ref, bv_ref,
             q_ref, k_ref, v_ref):
    xb = x_ref[0]
    q_ref[0, 0] = jnp.dot(xb, wq_ref[0], preferred_element_type=jnp.float32) + bq_ref[0]
    k_ref[0, 0] = jnp.dot(xb, wk_ref[0], preferred_element_type=jnp.float32) + bk_ref[0]
    v_ref[0, 0] = jnp.dot(xb, wv_ref[0], preferred_element_type=jnp.float32) + bv_ref[0]


def _qkv(x3, wq, wk, wv, bq, bk, bv):
    nq = C // CQ
    w_spec = pl.BlockSpec((1, D, DH), lambda b, cb, h: (h, 0, 0))
    b_spec = pl.BlockSpec((1, 1, DH), lambda b, cb, h: (h, 0, 0))
    o_spec = pl.BlockSpec((1, 1, CQ, DH), lambda b, cb, h: (b, h, cb, 0))
    o_shape = jax.ShapeDtypeStruct((B, H, C, DH), jnp.float32)
    return pl.pallas_call(
        _qkv_krn,
        grid=(B, nq, H),
        in_specs=[pl.BlockSpec((1, CQ, D), lambda b, cb, h: (b, cb, 0)),
                  w_spec, w_spec, w_spec, b_spec, b_spec, b_spec],
        out_specs=[o_spec, o_spec, o_spec],
        out_shape=[o_shape, o_shape, o_shape],
        compiler_params=pltpu.CompilerParams(
            dimension_semantics=("arbitrary", "arbitrary", "arbitrary")),
    )(x3, wq, wk, wv, bq, bk, bv)


def _attnp_krn(q_ref, k_ref, p_ref):
    q = q_ref[0, 0]
    k = k_ref[0, 0]
    p_ref[0, 0] = jax.lax.dot_general(q, k, (((1,), (1,)), ((), ())),
                                      preferred_element_type=jnp.float32) * (1.0 / 8.0)


def _attn_probs(q, k):
    nq = C // CQ
    kv_spec = pl.BlockSpec((1, 1, C, DH), lambda b, h, qb: (b, h, 0, 0))
    q_spec = pl.BlockSpec((1, 1, CQ, DH), lambda b, h, qb: (b, h, qb, 0))
    return pl.pallas_call(
        _attnp_krn,
        grid=(B, H, nq),
        in_specs=[q_spec, kv_spec],
        out_specs=pl.BlockSpec((1, 1, CQ, C), lambda b, h, qb: (b, h, qb, 0)),
        out_shape=jax.ShapeDtypeStruct((B, H, C, C), jnp.float32),
        compiler_params=pltpu.CompilerParams(
            dimension_semantics=("arbitrary", "arbitrary", "arbitrary")),
    )(q, k)


def _ln(r, g, b):
    m = jnp.mean(r, axis=-1, keepdims=True)
    var = jnp.mean((r - m) * (r - m), axis=-1, keepdims=True)
    return (r - m) / jnp.sqrt(var + EPS) * g + b


def _oproj_krn(x_ref, a_ref, wo_ref, bo_ref, g_ref, b_ref, wr_ref, br_ref,
               s_ref, lg_ref):
    a = jnp.dot(a_ref[...], wo_ref[...], preferred_element_type=jnp.float32) + bo_ref[...]
    s = _ln(x_ref[...] + a, g_ref[...], b_ref[...])
    s_ref[...] = s
    lg_ref[...] = jnp.dot(s, wr_ref[...],
                          preferred_element_type=jnp.float32) + br_ref[...]


def _oproj_ln_router(x, ao, wo, bo, g, b, wr, br):
    n = T // CQ
    full = lambda shape: pl.BlockSpec(shape, lambda i: (0,) * len(shape))
    row = pl.BlockSpec((CQ, D), lambda i: (i, 0))
    return pl.pallas_call(
        _oproj_krn,
        grid=(n,),
        in_specs=[row, row, full((D, D)), full((1, D)), full((1, D)),
                  full((1, D)), full((D, E)), full((1, E))],
        out_specs=[row, pl.BlockSpec((CQ, E), lambda i: (i, 0))],
        out_shape=[jax.ShapeDtypeStruct((T, D), jnp.float32),
                   jax.ShapeDtypeStruct((T, E), jnp.float32)],
        compiler_params=pltpu.CompilerParams(
            dimension_semantics=("arbitrary",)),
    )(x, ao, wo, bo, g, b, wr, br)


def _route_krn(lg_ref, ss_ref, sg_ref, gs_ref, cnt_ref):
    pid = pl.program_id(0)

    @pl.when(pid == 0)
    def _():
        cnt_ref[...] = jnp.zeros_like(cnt_ref)

    l = lg_ref[...]
    idx = jax.lax.broadcasted_iota(jnp.int32, (TB, E), 1).astype(jnp.float32)
    m1 = jnp.max(l, axis=1, keepdims=True)
    i1 = jnp.min(jnp.where(l == m1, idx, jnp.float32(E)), axis=1, keepdims=True)
    oh0 = (idx == i1).astype(jnp.float32)
    l2 = jnp.where(oh0 > 0.0, jnp.float32(-1e30), l)
    m2 = jnp.max(l2, axis=1, keepdims=True)
    i2 = jnp.min(jnp.where(l2 == m2, idx, jnp.float32(E)), axis=1, keepdims=True)
    oh1 = (idx == i2).astype(jnp.float32)
    g0 = 1.0 / (1.0 + jnp.exp(m2 - m1))
    g1 = 1.0 - g0

    ohsum = oh0 + oh1
    rr = jax.lax.broadcasted_iota(jnp.int32, (TB, TB), 0)
    cc = jax.lax.broadcasted_iota(jnp.int32, (TB, TB), 1)
    tri = (cc < rr).astype(jnp.float32)
    excl = jnp.dot(tri, ohsum, preferred_element_type=jnp.float32)
    carry = cnt_ref[0:1, 0:E]
    excl = excl + carry
    cnt_ref[0:1, 0:E] = carry + jnp.sum(ohsum, axis=0, keepdims=True)

    pos0 = jnp.sum(excl * oh0, axis=1, keepdims=True)
    pos1 = jnp.sum((excl + oh0) * oh1, axis=1, keepdims=True)
    keep0 = (pos0 < CAP).astype(jnp.float32)
    keep1 = (pos1 < CAP).astype(jnp.float32)
    p0c = jnp.minimum(pos0, jnp.float32(CAP - 1))
    p1c = jnp.minimum(pos1, jnp.float32(CAP - 1))
    s0 = i1 * CAP + p0c
    s1 = i2 * CAP + p1c
    sent = jnp.float32(E * CAP)
    ss0 = jnp.where(keep0 > 0.0, s0, sent)
    ss1 = jnp.where(keep1 > 0.0, s1, sent)
    ss_ref[...] = jnp.concatenate([ss0, ss1], axis=1).astype(jnp.int32)
    sg_ref[...] = jnp.concatenate([s0, s1], axis=1).astype(jnp.int32)
    gs_ref[...] = jnp.concatenate([g0 * keep0, g1 * keep1], axis=1)


def _route(logits):
    n = T // TB
    blk = lambda: pl.BlockSpec((TB, 2), lambda i: (i, 0))
    return pl.pallas_call(
        _route_krn,
        grid=(n,),
        in_specs=[pl.BlockSpec((TB, E), lambda i: (i, 0))],
        out_specs=[blk(), blk(), blk()],
        out_shape=[jax.ShapeDtypeStruct((T, 2), jnp.int32),
                   jax.ShapeDtypeStruct((T, 2), jnp.int32),
                   jax.ShapeDtypeStruct((T, 2), jnp.float32)],
        scratch_shapes=[pltpu.VMEM((E, 128), jnp.float32)],
        compiler_params=pltpu.CompilerParams(
            dimension_semantics=("arbitrary",)),
    )(logits)


def _inv_krn(ss_ref, inv_ref):
    def body(t, carry):
        inv_ref[ss_ref[0, t]] = t
        inv_ref[ss_ref[1, t]] = t
        return carry

    jax.lax.fori_loop(0, T, body, 0)


def _inv_scatter(slot_scatter):
    return pl.pallas_call(
        _inv_krn,
        grid=(1,),
        in_specs=[pl.BlockSpec(memory_space=pltpu.SMEM)],
        out_specs=pl.BlockSpec(memory_space=pltpu.SMEM),
        out_shape=jax.ShapeDtypeStruct((INV_PAD,), jnp.int32),
        compiler_params=pltpu.CompilerParams(
            dimension_semantics=("arbitrary",)),
    )(slot_scatter)


def _ffn_krn(x_ref, w1_ref, b1_ref, w2_ref, b2_ref, y_ref):
    h = jax.nn.gelu(jnp.dot(x_ref[...], w1_ref[0],
                            preferred_element_type=jnp.float32) + b1_ref[0])
    y_ref[...] = jnp.dot(h, w2_ref[0],
                         preferred_element_type=jnp.float32) + b2_ref[0]


def _ffn(disp, w1, b1, w2, b2):
    nc = CAP // CB
    b1 = b1.reshape(E, 1, FF)
    b2 = b2.reshape(E, 1, D)
    return pl.pallas_call(
        _ffn_krn,
        grid=(E, nc),
        in_specs=[pl.BlockSpec((CB, D), lambda e, cb: (e * nc + cb, 0)),
                  pl.BlockSpec((1, D, FF), lambda e, cb: (e, 0, 0)),
                  pl.BlockSpec((1, 1, FF), lambda e, cb: (e, 0, 0)),
                  pl.BlockSpec((1, FF, D), lambda e, cb: (e, 0, 0)),
                  pl.BlockSpec((1, 1, D), lambda e, cb: (e, 0, 0))],
        out_specs=pl.BlockSpec((CB, D), lambda e, cb: (e * nc + cb, 0)),
        out_shape=jax.ShapeDtypeStruct((E * CAP, D), jnp.float32),
        compiler_params=pltpu.CompilerParams(
            dimension_semantics=("arbitrary", "arbitrary")),
    )(disp, w1, b1, w2, b2)


def _combine_krn(s_ref, y_ref, gs_ref, g_ref, b_ref, o_ref):
    y = y_ref[...]
    y0 = y[:, :D]
    y1 = y[:, D:]
    g0 = gs_ref[:, 0:1]
    g1 = gs_ref[:, 1:2]
    r = s_ref[...] + y0 * g0 + y1 * g1
    o_ref[...] = _ln(r, g_ref[...], b_ref[...])


def _combine_ln(s, yg2, gscale, g, b):
    n = T // CQ
    full = lambda shape: pl.BlockSpec(shape, lambda i: (0,) * len(shape))
    row = pl.BlockSpec((CQ, D), lambda i: (i, 0))
    return pl.pallas_call(
        _combine_krn,
        grid=(n,),
        in_specs=[row, pl.BlockSpec((CQ, 2 * D), lambda i: (i, 0)),
                  pl.BlockSpec((CQ, 2), lambda i: (i, 0)),
                  full((1, D)), full((1, D))],
        out_specs=row,
        out_shape=jax.ShapeDtypeStruct((T, D), jnp.float32),
        compiler_params=pltpu.CompilerParams(
            dimension_semantics=("arbitrary",)),
    )(s, yg2, gscale, g, b)


def kernel(src_BC, src_mask_B11C, steps_B1, tok_emb, pos_emb, step_emb,
           Wq, bq, Wk, bk, Wv, bv, Wo, bo, ln1_g, ln1_b, ln2_g, ln2_b,
           Wr, br, W1, b1, W2, b2):
    del src_mask_B11C

    src_flat = src_BC.reshape(-1)
    step_add = steps_B1 * step_emb
    x = _embed(src_flat, tok_emb, pos_emb[:C], step_add)

    for l in range(L):
        wq = Wq[l].reshape(D, H, DH).transpose(1, 0, 2)
        wk = Wk[l].reshape(D, H, DH).transpose(1, 0, 2)
        wv = Wv[l].reshape(D, H, DH).transpose(1, 0, 2)
        bq_h = bq[l].reshape(H, 1, DH)
        bk_h = bk[l].reshape(H, 1, DH)
        bv_h = bv[l].reshape(H, 1, DH)

        x3 = x.reshape(B, C, D)
        q, k, v = _qkv(x3, wq, wk, wv, bq_h, bk_h, bv_h)
        energy = _attn_probs(q, k)
        attnp = jax.nn.softmax(energy, axis=-1)
        ao = jnp.einsum('bhqk,bhkd->bhqd', attnp, v)
        ao = ao.transpose(0, 2, 1, 3).reshape(T, D)

        s, logits = _oproj_ln_router(
            x, ao, Wo[l], bo[l].reshape(1, D), ln1_g[l].reshape(1, D),
            ln1_b[l].reshape(1, D), Wr[l], br[l].reshape(1, E))

        slot_scatter, slot_gather, gscale = _route(logits)
        inv = _inv_scatter(slot_scatter.T)
        disp = _row_gather(s, inv, E * CAP, T - 1)
        yb = _ffn(disp, W1[l], b1[l], W2[l], b2[l])

        sg_flat = slot_gather.reshape(-1)
        yg = _row_gather(yb, sg_flat, T * TOPK, E * CAP - 1)
        yg2 = yg.reshape(T, 2 * D)
        x = _combine_ln(s, yg2, gscale, ln2_g[l].reshape(1, D),
                        ln2_b[l].reshape(1, D))

    return x.reshape(B, C, D)


# speedup vs baseline: 1.2714x; 1.2714x over previous
import math

import jax
import numpy as np
import jax.numpy as jnp
from jax.experimental import pallas as pl
from jax.experimental.pallas import tpu as pltpu

B, C, D, H, L = 2, 2048, 768, 12, 2
E, TOPK, FF, V = 8, 2, 3072, 32000
DH = D // H
T = B * C
CAP = (TOPK * T) // E
EPS = 1e-5
SCALE = float(np.sqrt(np.float32(D)))
GR = 16
TB = 256
CQ = 512
CA = 2048
CB = 256
KC = 256
INV_PAD = E * CAP + 8


def _gather_krn(idx_ref, *refs):
    del idx_ref
    ins, out_ref = refs[:GR], refs[GR]
    out_ref[...] = jnp.concatenate([r[0] for r in ins], axis=0)


def _row_gather(table, idx, n_rows, clip_hi):
    def imap(j):
        def f(i, idx_ref):
            v = idx_ref[i * GR + j]
            return (jnp.minimum(jnp.maximum(v, 0), clip_hi), 0, 0)
        return f

    grid_spec = pltpu.PrefetchScalarGridSpec(
        num_scalar_prefetch=1,
        grid=(n_rows // GR,),
        in_specs=[pl.BlockSpec((1, 1, D), imap(j)) for j in range(GR)],
        out_specs=pl.BlockSpec((GR, D), lambda i, idx_ref: (i, 0)),
    )
    return pl.pallas_call(
        _gather_krn,
        grid_spec=grid_spec,
        out_shape=jax.ShapeDtypeStruct((n_rows, D), jnp.float32),
        compiler_params=pltpu.CompilerParams(
            dimension_semantics=("arbitrary",)),
    )(idx, *([table.reshape(-1, 1, D)] * GR))


def _embed_krn(idx_ref, *refs):
    del idx_ref
    toks, pos_ref, step_ref, out_ref = refs[:GR], refs[GR], refs[GR + 1], refs[GR + 2]
    rows = jnp.concatenate([r[0] for r in toks], axis=0)
    out_ref[...] = rows * SCALE + pos_ref[...] + step_ref[0]


def _embed(src_flat, tok_emb, pos_c, step_add):
    def imap(j):
        def f(i, idx_ref):
            return (idx_ref[i * GR + j], 0, 0)
        return f

    cpg = C // GR
    grid_spec = pltpu.PrefetchScalarGridSpec(
        num_scalar_prefetch=1,
        grid=(T // GR,),
        in_specs=[pl.BlockSpec((1, 1, D), imap(j)) for j in range(GR)]
        + [pl.BlockSpec((GR, D), lambda i, idx_ref: (i % cpg, 0)),
           pl.BlockSpec((1, 1, D), lambda i, idx_ref: (i // cpg, 0, 0))],
        out_specs=pl.BlockSpec((GR, D), lambda i, idx_ref: (i, 0)),
    )
    return pl.pallas_call(
        _embed_krn,
        grid_spec=grid_spec,
        out_shape=jax.ShapeDtypeStruct((T, D), jnp.float32),
        compiler_params=pltpu.CompilerParams(
            dimension_semantics=("arbitrary",)),
    )(src_flat, *([tok_emb.reshape(V, 1, D)] * GR), pos_c,
      step_add.reshape(B, 1, D))


def _qkv_krn(x_ref, wq_ref, wk_ref, wv_ref, bq_ref, bk_ref, bv_ref,
             q_ref, k_ref, v_ref):
    xb = x_ref[0]
    q_ref[0, 0] = jnp.dot(xb, wq_ref[0], preferred_element_type=jnp.float32) + bq_ref[0]
    k_ref[0, 0] = jnp.dot(xb, wk_ref[0], preferred_element_type=jnp.float32) + bk_ref[0]
    v_ref[0, 0] = jnp.dot(xb, wv_ref[0], preferred_element_type=jnp.float32) + bv_ref[0]


def _qkv(x3, wq, wk, wv, bq, bk, bv):
    nq = C // CQ
    w_spec = pl.BlockSpec((1, D, DH), lambda b, cb, h: (h, 0, 0))
    b_spec = pl.BlockSpec((1, 1, DH), lambda b, cb, h: (h, 0, 0))
    o_spec = pl.BlockSpec((1, 1, CQ, DH), lambda b, cb, h: (b, h, cb, 0))
    o_shape = jax.ShapeDtypeStruct((B, H, C, DH), jnp.float32)
    return pl.pallas_call(
        _qkv_krn,
        grid=(B, nq, H),
        in_specs=[pl.BlockSpec((1, CQ, D), lambda b, cb, h: (b, cb, 0)),
                  w_spec, w_spec, w_spec, b_spec, b_spec, b_spec],
        out_specs=[o_spec, o_spec, o_spec],
        out_shape=[o_shape, o_shape, o_shape],
        compiler_params=pltpu.CompilerParams(
            dimension_semantics=("arbitrary", "arbitrary", "arbitrary")),
    )(x3, wq, wk, wv, bq, bk, bv)


def _attnp_krn(q_ref, k_ref, p_ref):
    q = q_ref[0, 0]
    k = k_ref[0, 0]
    p_ref[0, 0] = jax.lax.dot_general(q, k, (((1,), (1,)), ((), ())),
                                      preferred_element_type=jnp.float32) * (1.0 / 8.0)


def _attn_probs(q, k):
    nq = C // CQ
    kv_spec = pl.BlockSpec((1, 1, C, DH), lambda b, h, qb: (b, h, 0, 0))
    q_spec = pl.BlockSpec((1, 1, CQ, DH), lambda b, h, qb: (b, h, qb, 0))
    return pl.pallas_call(
        _attnp_krn,
        grid=(B, H, nq),
        in_specs=[q_spec, kv_spec],
        out_specs=pl.BlockSpec((1, 1, CQ, C), lambda b, h, qb: (b, h, qb, 0)),
        out_shape=jax.ShapeDtypeStruct((B, H, C, C), jnp.float32),
        compiler_params=pltpu.CompilerParams(
            dimension_semantics=("arbitrary", "arbitrary", "arbitrary")),
    )(q, k)


def _ln(r, g, b):
    m = jnp.mean(r, axis=-1, keepdims=True)
    var = jnp.mean((r - m) * (r - m), axis=-1, keepdims=True)
    return (r - m) / jnp.sqrt(var + EPS) * g + b


def _oproj_krn(x_ref, a_ref, wo_ref, bo_ref, g_ref, b_ref, wr_ref, br_ref,
               s_ref, lg_ref):
    a = jnp.dot(a_ref[...], wo_ref[...], preferred_element_type=jnp.float32) + bo_ref[...]
    s = _ln(x_ref[...] + a, g_ref[...], b_ref[...])
    s_ref[...] = s
    lg_ref[...] = jnp.dot(s, wr_ref[...],
                          preferred_element_type=jnp.float32) + br_ref[...]


def _oproj_ln_router(x, ao, wo, bo, g, b, wr, br):
    n = T // CQ
    full = lambda shape: pl.BlockSpec(shape, lambda i: (0,) * len(shape))
    row = pl.BlockSpec((CQ, D), lambda i: (i, 0))
    return pl.pallas_call(
        _oproj_krn,
        grid=(n,),
        in_specs=[row, row, full((D, D)), full((1, D)), full((1, D)),
                  full((1, D)), full((D, E)), full((1, E))],
        out_specs=[row, pl.BlockSpec((CQ, E), lambda i: (i, 0))],
        out_shape=[jax.ShapeDtypeStruct((T, D), jnp.float32),
                   jax.ShapeDtypeStruct((T, E), jnp.float32)],
        compiler_params=pltpu.CompilerParams(
            dimension_semantics=("arbitrary",)),
    )(x, ao, wo, bo, g, b, wr, br)


def _route_krn(lg_ref, ss_ref, sg_ref, gs_ref, cnt_ref):
    pid = pl.program_id(0)

    @pl.when(pid == 0)
    def _():
        cnt_ref[...] = jnp.zeros_like(cnt_ref)

    l = lg_ref[...]
    idx = jax.lax.broadcasted_iota(jnp.int32, (TB, E), 1).astype(jnp.float32)
    m1 = jnp.max(l, axis=1, keepdims=True)
    i1 = jnp.min(jnp.where(l == m1, idx, jnp.float32(E)), axis=1, keepdims=True)
    oh0 = (idx == i1).astype(jnp.float32)
    l2 = jnp.where(oh0 > 0.0, jnp.float32(-1e30), l)
    m2 = jnp.max(l2, axis=1, keepdims=True)
    i2 = jnp.min(jnp.where(l2 == m2, idx, jnp.float32(E)), axis=1, keepdims=True)
    oh1 = (idx == i2).astype(jnp.float32)
    g0 = 1.0 / (1.0 + jnp.exp(m2 - m1))
    g1 = 1.0 - g0

    ohsum = oh0 + oh1
    rr = jax.lax.broadcasted_iota(jnp.int32, (TB, TB), 0)
    cc = jax.lax.broadcasted_iota(jnp.int32, (TB, TB), 1)
    tri = (cc < rr).astype(jnp.float32)
    excl = jnp.dot(tri, ohsum, preferred_element_type=jnp.float32)
    carry = cnt_ref[0:1, 0:E]
    excl = excl + carry
    cnt_ref[0:1, 0:E] = carry + jnp.sum(ohsum, axis=0, keepdims=True)

    pos0 = jnp.sum(excl * oh0, axis=1, keepdims=True)
    pos1 = jnp.sum((excl + oh0) * oh1, axis=1, keepdims=True)
    keep0 = (pos0 < CAP).astype(jnp.float32)
    keep1 = (pos1 < CAP).astype(jnp.float32)
    p0c = jnp.minimum(pos0, jnp.float32(CAP - 1))
    p1c = jnp.minimum(pos1, jnp.float32(CAP - 1))
    s0 = i1 * CAP + p0c
    s1 = i2 * CAP + p1c
    sent = jnp.float32(E * CAP)
    ss0 = jnp.where(keep0 > 0.0, s0, sent)
    ss1 = jnp.where(keep1 > 0.0, s1, sent)
    ss_ref[...] = jnp.concatenate([ss0, ss1], axis=1).astype(jnp.int32)
    sg_ref[...] = jnp.concatenate([s0, s1], axis=1).astype(jnp.int32)
    gs_ref[...] = jnp.concatenate([g0 * keep0, g1 * keep1], axis=1)


def _route(logits):
    n = T // TB
    blk = lambda: pl.BlockSpec((TB, 2), lambda i: (i, 0))
    return pl.pallas_call(
        _route_krn,
        grid=(n,),
        in_specs=[pl.BlockSpec((TB, E), lambda i: (i, 0))],
        out_specs=[blk(), blk(), blk()],
        out_shape=[jax.ShapeDtypeStruct((T, 2), jnp.int32),
                   jax.ShapeDtypeStruct((T, 2), jnp.int32),
                   jax.ShapeDtypeStruct((T, 2), jnp.float32)],
        scratch_shapes=[pltpu.VMEM((E, 128), jnp.float32)],
        compiler_params=pltpu.CompilerParams(
            dimension_semantics=("arbitrary",)),
    )(logits)


def _inv_krn(ss_ref, inv_ref):
    def body(t, carry):
        inv_ref[ss_ref[0, t]] = t
        inv_ref[ss_ref[1, t]] = t
        return carry

    jax.lax.fori_loop(0, T, body, 0)


def _inv_scatter(slot_scatter):
    return pl.pallas_call(
        _inv_krn,
        grid=(1,),
        in_specs=[pl.BlockSpec(memory_space=pltpu.SMEM)],
        out_specs=pl.BlockSpec(memory_space=pltpu.SMEM),
        out_shape=jax.ShapeDtypeStruct((INV_PAD,), jnp.int32),
        compiler_params=pltpu.CompilerParams(
            dimension_semantics=("arbitrary",)),
    )(slot_scatter)


def _ffn_krn(x_ref, w1_ref, b1_ref, w2_ref, b2_ref, y_ref):
    h = jax.nn.gelu(jnp.dot(x_ref[...], w1_ref[0],
                            preferred_element_type=jnp.float32) + b1_ref[0])
    y_ref[...] = jnp.dot(h, w2_ref[0],
                         preferred_element_type=jnp.float32) + b2_ref[0]


def _ffn(disp, w1, b1, w2, b2):
    nc = CAP // CB
    b1 = b1.reshape(E, 1, FF)
    b2 = b2.reshape(E, 1, D)
    return pl.pallas_call(
        _ffn_krn,
        grid=(E, nc),
        in_specs=[pl.BlockSpec((CB, D), lambda e, cb: (e * nc + cb, 0)),
                  pl.BlockSpec((1, D, FF), lambda e, cb: (e, 0, 0)),
                  pl.BlockSpec((1, 1, FF), lambda e, cb: (e, 0, 0)),
                  pl.BlockSpec((1, FF, D), lambda e, cb: (e, 0, 0)),
                  pl.BlockSpec((1, 1, D), lambda e, cb: (e, 0, 0))],
        out_specs=pl.BlockSpec((CB, D), lambda e, cb: (e * nc + cb, 0)),
        out_shape=jax.ShapeDtypeStruct((E * CAP, D), jnp.float32),
        compiler_params=pltpu.CompilerParams(
            dimension_semantics=("arbitrary", "arbitrary")),
    )(disp, w1, b1, w2, b2)


def _combine_krn(s_ref, y_ref, gs_ref, g_ref, b_ref, o_ref):
    y = y_ref[...]
    y0 = y[:, :D]
    y1 = y[:, D:]
    g0 = gs_ref[:, 0:1]
    g1 = gs_ref[:, 1:2]
    r = s_ref[...] + y0 * g0 + y1 * g1
    o_ref[...] = _ln(r, g_ref[...], b_ref[...])


def _combine_ln(s, yg2, gscale, g, b):
    n = T // CQ
    full = lambda shape: pl.BlockSpec(shape, lambda i: (0,) * len(shape))
    row = pl.BlockSpec((CQ, D), lambda i: (i, 0))
    return pl.pallas_call(
        _combine_krn,
        grid=(n,),
        in_specs=[row, pl.BlockSpec((CQ, 2 * D), lambda i: (i, 0)),
                  pl.BlockSpec((CQ, 2), lambda i: (i, 0)),
                  full((1, D)), full((1, D))],
        out_specs=row,
        out_shape=jax.ShapeDtypeStruct((T, D), jnp.float32),
        compiler_params=pltpu.CompilerParams(
            dimension_semantics=("arbitrary",)),
    )(s, yg2, gscale, g, b)


def kernel(src_BC, src_mask_B11C, steps_B1, tok_emb, pos_emb, step_emb,
           Wq, bq, Wk, bk, Wv, bv, Wo, bo, ln1_g, ln1_b, ln2_g, ln2_b,
           Wr, br, W1, b1, W2, b2):
    del src_mask_B11C

    src_flat = src_BC.reshape(-1)
    step_add = steps_B1 * step_emb
    x = _embed(src_flat, tok_emb, pos_emb[:C], step_add)

    for l in range(L):
        wq = Wq[l].reshape(D, H, DH).transpose(1, 0, 2)
        wk = Wk[l].reshape(D, H, DH).transpose(1, 0, 2)
        wv = Wv[l].reshape(D, H, DH).transpose(1, 0, 2)
        bq_h = bq[l].reshape(H, 1, DH)
        bk_h = bk[l].reshape(H, 1, DH)
        bv_h = bv[l].reshape(H, 1, DH)

        x3 = x.reshape(B, C, D)
        q, k, v = _qkv(x3, wq, wk, wv, bq_h, bk_h, bv_h)
        energy = _attn_probs(q, k)
        attnp = jax.nn.softmax(energy, axis=-1)
        ao = jnp.einsum('bhqk,bhkd->bhqd', attnp, v)
        ao = ao.transpose(0, 2, 1, 3).reshape(T, D)

        s, logits = _oproj_ln_router(
            x, ao, Wo[l], bo[l].reshape(1, D), ln1_g[l].reshape(1, D),
            ln1_b[l].reshape(1, D), Wr[l], br[l].reshape(1, E))

        slot_scatter, slot_gather, gscale = _route(logits)
        inv = _inv_scatter(slot_scatter.T)
        disp = _row_gather(s, inv, E * CAP, T - 1)
        yb = _ffn(disp, W1[l], b1[l], W2[l], b2[l])

        sg_flat = slot_gather.reshape(-1)
        yg = _row_gather(yb, sg_flat, T * TOPK, E * CAP - 1)
        yg2 = yg.reshape(T, 2 * D)
        x = _combine_ln(s, yg2, gscale, ln2_g[l].reshape(1, D),
                        ln2_b[l].reshape(1, D))

    return x.reshape(B, C, D)


# speedup vs baseline: 1.5605x; 1.2274x over previous
import math

import jax
import numpy as np
import jax.numpy as jnp
from jax.experimental import pallas as pl
from jax.experimental.pallas import tpu as pltpu

B, C, D, H, L = 2, 2048, 768, 12, 2
E, TOPK, FF, V = 8, 2, 3072, 32000
DH = D // H
T = B * C
CAP = (TOPK * T) // E
EPS = 1e-5
SCALE = float(np.sqrt(np.float32(D)))
GR = 32
TB = 256
CQ = 512
CA = 2048
CB = 256
KC = 256
INV_PAD = E * CAP + 8


def _gather_krn(idx_ref, *refs):
    del idx_ref
    ins, out_ref = refs[:GR], refs[GR]
    out_ref[...] = jnp.concatenate([r[0] for r in ins], axis=0)


def _row_gather(table, idx, n_rows, clip_hi):
    def imap(j):
        def f(i, idx_ref):
            v = idx_ref[i * GR + j]
            return (jnp.minimum(jnp.maximum(v, 0), clip_hi), 0, 0)
        return f

    grid_spec = pltpu.PrefetchScalarGridSpec(
        num_scalar_prefetch=1,
        grid=(n_rows // GR,),
        in_specs=[pl.BlockSpec((1, 1, D), imap(j)) for j in range(GR)],
        out_specs=pl.BlockSpec((GR, D), lambda i, idx_ref: (i, 0)),
    )
    return pl.pallas_call(
        _gather_krn,
        grid_spec=grid_spec,
        out_shape=jax.ShapeDtypeStruct((n_rows, D), jnp.float32),
        compiler_params=pltpu.CompilerParams(
            dimension_semantics=("arbitrary",)),
    )(idx, *([table.reshape(-1, 1, D)] * GR))


def _embed_krn(idx_ref, *refs):
    del idx_ref
    toks, pos_ref, step_ref, out_ref = refs[:GR], refs[GR], refs[GR + 1], refs[GR + 2]
    rows = jnp.concatenate([r[0] for r in toks], axis=0)
    out_ref[...] = rows * SCALE + pos_ref[...] + step_ref[0]


def _embed(src_flat, tok_emb, pos_c, step_add):
    def imap(j):
        def f(i, idx_ref):
            return (idx_ref[i * GR + j], 0, 0)
        return f

    cpg = C // GR
    grid_spec = pltpu.PrefetchScalarGridSpec(
        num_scalar_prefetch=1,
        grid=(T // GR,),
        in_specs=[pl.BlockSpec((1, 1, D), imap(j)) for j in range(GR)]
        + [pl.BlockSpec((GR, D), lambda i, idx_ref: (i % cpg, 0)),
           pl.BlockSpec((1, 1, D), lambda i, idx_ref: (i // cpg, 0, 0))],
        out_specs=pl.BlockSpec((GR, D), lambda i, idx_ref: (i, 0)),
    )
    return pl.pallas_call(
        _embed_krn,
        grid_spec=grid_spec,
        out_shape=jax.ShapeDtypeStruct((T, D), jnp.float32),
        compiler_params=pltpu.CompilerParams(
            dimension_semantics=("arbitrary",)),
    )(src_flat, *([tok_emb.reshape(V, 1, D)] * GR), pos_c,
      step_add.reshape(B, 1, D))


def _qkvf_krn(x_ref, w_ref, b_ref, o_ref):
    o_ref[...] = jnp.dot(x_ref[...], w_ref[...],
                         preferred_element_type=jnp.float32) + b_ref[...]


def _qkv_fused(x, wqkv, bqkv):
    n = T // CQ
    return pl.pallas_call(
        _qkvf_krn,
        grid=(n,),
        in_specs=[pl.BlockSpec((CQ, D), lambda i: (i, 0)),
                  pl.BlockSpec((D, 3 * D), lambda i: (0, 0)),
                  pl.BlockSpec((1, 3 * D), lambda i: (0, 0))],
        out_specs=pl.BlockSpec((CQ, 3 * D), lambda i: (i, 0)),
        out_shape=jax.ShapeDtypeStruct((T, 3 * D), jnp.float32),
        compiler_params=pltpu.CompilerParams(
            dimension_semantics=("arbitrary",)),
    )(x, wqkv, bqkv)


def _attnp_krn(q_ref, k_ref, p_ref):
    q2 = q_ref[0]
    k2 = k_ref[0]
    for j in range(2):
        q = q2[:, j * DH:(j + 1) * DH]
        k = k2[:, j * DH:(j + 1) * DH]
        p_ref[0, j] = jax.lax.dot_general(
            q, k, (((1,), (1,)), ((), ())),
            preferred_element_type=jnp.float32) * (1.0 / 8.0)


def _attn_probs(qkv3):
    nq = C // CQ
    q_spec = pl.BlockSpec((1, CQ, 2 * DH), lambda b, h2, qb: (b, qb, h2))
    k_spec = pl.BlockSpec((1, C, 2 * DH), lambda b, h2, qb: (b, 0, H // 2 + h2))
    return pl.pallas_call(
        _attnp_krn,
        grid=(B, H // 2, nq),
        in_specs=[q_spec, k_spec],
        out_specs=pl.BlockSpec((1, 2, CQ, C), lambda b, h2, qb: (b, h2, qb, 0)),
        out_shape=jax.ShapeDtypeStruct((B, H, C, C), jnp.float32),
        compiler_params=pltpu.CompilerParams(
            dimension_semantics=("arbitrary", "arbitrary", "arbitrary")),
    )(qkv3, qkv3)


def _ln(r, g, b):
    m = jnp.mean(r, axis=-1, keepdims=True)
    var = jnp.mean((r - m) * (r - m), axis=-1, keepdims=True)
    return (r - m) / jnp.sqrt(var + EPS) * g + b


def _oproj_krn(x_ref, a_ref, wo_ref, bo_ref, g_ref, b_ref, wr_ref, br_ref,
               s_ref, lg_ref):
    a = jnp.dot(a_ref[...], wo_ref[...], preferred_element_type=jnp.float32) + bo_ref[...]
    s = _ln(x_ref[...] + a, g_ref[...], b_ref[...])
    s_ref[...] = s
    lg_ref[...] = jnp.dot(s, wr_ref[...],
                          preferred_element_type=jnp.float32) + br_ref[...]


def _oproj_ln_router(x, ao, wo, bo, g, b, wr, br):
    n = T // CQ
    full = lambda shape: pl.BlockSpec(shape, lambda i: (0,) * len(shape))
    row = pl.BlockSpec((CQ, D), lambda i: (i, 0))
    return pl.pallas_call(
        _oproj_krn,
        grid=(n,),
        in_specs=[row, row, full((D, D)), full((1, D)), full((1, D)),
                  full((1, D)), full((D, E)), full((1, E))],
        out_specs=[row, pl.BlockSpec((CQ, E), lambda i: (i, 0))],
        out_shape=[jax.ShapeDtypeStruct((T, D), jnp.float32),
                   jax.ShapeDtypeStruct((T, E), jnp.float32)],
        compiler_params=pltpu.CompilerParams(
            dimension_semantics=("arbitrary",)),
    )(x, ao, wo, bo, g, b, wr, br)


def _route_krn(lg_ref, ss_ref, sg_ref, gs_ref, cnt_ref):
    pid = pl.program_id(0)

    @pl.when(pid == 0)
    def _():
        cnt_ref[...] = jnp.zeros_like(cnt_ref)

    l = lg_ref[...]
    idx = jax.lax.broadcasted_iota(jnp.int32, (TB, E), 1).astype(jnp.float32)
    m1 = jnp.max(l, axis=1, keepdims=True)
    i1 = jnp.min(jnp.where(l == m1, idx, jnp.float32(E)), axis=1, keepdims=True)
    oh0 = (idx == i1).astype(jnp.float32)
    l2 = jnp.where(oh0 > 0.0, jnp.float32(-1e30), l)
    m2 = jnp.max(l2, axis=1, keepdims=True)
    i2 = jnp.min(jnp.where(l2 == m2, idx, jnp.float32(E)), axis=1, keepdims=True)
    oh1 = (idx == i2).astype(jnp.float32)
    g0 = 1.0 / (1.0 + jnp.exp(m2 - m1))
    g1 = 1.0 - g0

    ohsum = oh0 + oh1
    rr = jax.lax.broadcasted_iota(jnp.int32, (TB, TB), 0)
    cc = jax.lax.broadcasted_iota(jnp.int32, (TB, TB), 1)
    tri = (cc < rr).astype(jnp.float32)
    excl = jnp.dot(tri, ohsum, preferred_element_type=jnp.float32)
    carry = cnt_ref[0:1, 0:E]
    excl = excl + carry
    cnt_ref[0:1, 0:E] = carry + jnp.sum(ohsum, axis=0, keepdims=True)

    pos0 = jnp.sum(excl * oh0, axis=1, keepdims=True)
    pos1 = jnp.sum((excl + oh0) * oh1, axis=1, keepdims=True)
    keep0 = (pos0 < CAP).astype(jnp.float32)
    keep1 = (pos1 < CAP).astype(jnp.float32)
    p0c = jnp.minimum(pos0, jnp.float32(CAP - 1))
    p1c = jnp.minimum(pos1, jnp.float32(CAP - 1))
    s0 = i1 * CAP + p0c
    s1 = i2 * CAP + p1c
    sent = jnp.float32(E * CAP)
    ss0 = jnp.where(keep0 > 0.0, s0, sent)
    ss1 = jnp.where(keep1 > 0.0, s1, sent)
    ss_ref[...] = jnp.concatenate([ss0, ss1], axis=1).astype(jnp.int32)
    sg_ref[...] = jnp.concatenate([s0, s1], axis=1).astype(jnp.int32)
    gs_ref[...] = jnp.concatenate([g0 * keep0, g1 * keep1], axis=1)


def _route(logits):
    n = T // TB
    blk = lambda: pl.BlockSpec((TB, 2), lambda i: (i, 0))
    return pl.pallas_call(
        _route_krn,
        grid=(n,),
        in_specs=[pl.BlockSpec((TB, E), lambda i: (i, 0))],
        out_specs=[blk(), blk(), blk()],
        out_shape=[jax.ShapeDtypeStruct((T, 2), jnp.int32),
                   jax.ShapeDtypeStruct((T, 2), jnp.int32),
                   jax.ShapeDtypeStruct((T, 2), jnp.float32)],
        scratch_shapes=[pltpu.VMEM((E, 128), jnp.float32)],
        compiler_params=pltpu.CompilerParams(
            dimension_semantics=("arbitrary",)),
    )(logits)


def _inv_krn(ss_ref, inv_ref):
    def body(t, carry):
        inv_ref[ss_ref[0, t]] = t
        inv_ref[ss_ref[1, t]] = t
        return carry

    jax.lax.fori_loop(0, T, body, 0)


def _inv_scatter(slot_scatter):
    return pl.pallas_call(
        _inv_krn,
        grid=(1,),
        in_specs=[pl.BlockSpec(memory_space=pltpu.SMEM)],
        out_specs=pl.BlockSpec(memory_space=pltpu.SMEM),
        out_shape=jax.ShapeDtypeStruct((INV_PAD,), jnp.int32),
        compiler_params=pltpu.CompilerParams(
            dimension_semantics=("arbitrary",)),
    )(slot_scatter)


def _ffn_krn(x_ref, w1_ref, b1_ref, w2_ref, b2_ref, y_ref):
    h = jax.nn.gelu(jnp.dot(x_ref[...], w1_ref[0],
                            preferred_element_type=jnp.float32) + b1_ref[0])
    y_ref[...] = jnp.dot(h, w2_ref[0],
                         preferred_element_type=jnp.float32) + b2_ref[0]


def _ffn(disp, w1, b1, w2, b2):
    nc = CAP // CB
    b1 = b1.reshape(E, 1, FF)
    b2 = b2.reshape(E, 1, D)
    return pl.pallas_call(
        _ffn_krn,
        grid=(E, nc),
        in_specs=[pl.BlockSpec((CB, D), lambda e, cb: (e * nc + cb, 0)),
                  pl.BlockSpec((1, D, FF), lambda e, cb: (e, 0, 0)),
                  pl.BlockSpec((1, 1, FF), lambda e, cb: (e, 0, 0)),
                  pl.BlockSpec((1, FF, D), lambda e, cb: (e, 0, 0)),
                  pl.BlockSpec((1, 1, D), lambda e, cb: (e, 0, 0))],
        out_specs=pl.BlockSpec((CB, D), lambda e, cb: (e * nc + cb, 0)),
        out_shape=jax.ShapeDtypeStruct((E * CAP, D), jnp.float32),
        compiler_params=pltpu.CompilerParams(
            dimension_semantics=("arbitrary", "arbitrary")),
    )(disp, w1, b1, w2, b2)


def _combine_krn(s_ref, y_ref, gs_ref, g_ref, b_ref, o_ref):
    y = y_ref[...]
    y0 = y[:, :D]
    y1 = y[:, D:]
    g0 = gs_ref[:, 0:1]
    g1 = gs_ref[:, 1:2]
    r = s_ref[...] + y0 * g0 + y1 * g1
    o_ref[...] = _ln(r, g_ref[...], b_ref[...])


def _combine_ln(s, yg2, gscale, g, b):
    n = T // CQ
    full = lambda shape: pl.BlockSpec(shape, lambda i: (0,) * len(shape))
    row = pl.BlockSpec((CQ, D), lambda i: (i, 0))
    return pl.pallas_call(
        _combine_krn,
        grid=(n,),
        in_specs=[row, pl.BlockSpec((CQ, 2 * D), lambda i: (i, 0)),
                  pl.BlockSpec((CQ, 2), lambda i: (i, 0)),
                  full((1, D)), full((1, D))],
        out_specs=row,
        out_shape=jax.ShapeDtypeStruct((T, D), jnp.float32),
        compiler_params=pltpu.CompilerParams(
            dimension_semantics=("arbitrary",)),
    )(s, yg2, gscale, g, b)


def kernel(src_BC, src_mask_B11C, steps_B1, tok_emb, pos_emb, step_emb,
           Wq, bq, Wk, bk, Wv, bv, Wo, bo, ln1_g, ln1_b, ln2_g, ln2_b,
           Wr, br, W1, b1, W2, b2):
    del src_mask_B11C

    src_flat = src_BC.reshape(-1)
    step_add = steps_B1 * step_emb
    x = _embed(src_flat, tok_emb, pos_emb[:C], step_add)

    for l in range(L):
        wqkv = jnp.concatenate([Wq[l], Wk[l], Wv[l]], axis=1)
        bqkv = jnp.concatenate([bq[l], bk[l], bv[l]]).reshape(1, 3 * D)

        qkv = _qkv_fused(x, wqkv, bqkv)
        qkv3 = qkv.reshape(B, C, 3 * D)
        energy = _attn_probs(qkv3)
        attnp = jax.nn.softmax(energy, axis=-1)
        v = qkv3[:, :, 2 * D:].reshape(B, C, H, DH).transpose(0, 2, 1, 3)
        ao = jnp.einsum('bhqk,bhkd->bhqd', attnp, v)
        ao = ao.transpose(0, 2, 1, 3).reshape(T, D)

        s, logits = _oproj_ln_router(
            x, ao, Wo[l], bo[l].reshape(1, D), ln1_g[l].reshape(1, D),
            ln1_b[l].reshape(1, D), Wr[l], br[l].reshape(1, E))

        slot_scatter, slot_gather, gscale = _route(logits)
        inv = _inv_scatter(slot_scatter.T)
        disp = _row_gather(s, inv, E * CAP, T - 1)
        yb = _ffn(disp, W1[l], b1[l], W2[l], b2[l])

        sg_flat = slot_gather.reshape(-1)
        yg = _row_gather(yb, sg_flat, T * TOPK, E * CAP - 1)
        yg2 = yg.reshape(T, 2 * D)
        x = _combine_ln(s, yg2, gscale, ln2_g[l].reshape(1, D),
                        ln2_b[l].reshape(1, D))

    return x.reshape(B, C, D)
